# baseline (speedup 1.0000x reference)
import functools

import jax
import jax.numpy as jnp
from jax import lax
from jax.experimental import pallas as pl
from jax.experimental.pallas import tpu as pltpu
from jax.experimental.pallas import tpu_sc as plsc

N = 10000
E = 320000
D = 128
N_PAD = 10240

_info = plsc.get_sparse_core_info()
NC = _info.num_cores
NS = _info.num_subcores
NW = NC * NS
EPW = E // NW
BLK = 80
NBLK = EPW // BLK
GRP = 25
NGRP = NBLK // GRP
ROWS_PT = N_PAD // NS
RCHUNK = BLK
NCHUNK = ROWS_PT // RCHUNK


def _sc_aggregate(h_pad, src3, dst3):
    mesh = plsc.VectorSubcoreMesh(core_axis_name="c", subcore_axis_name="s")

    @functools.partial(
        pl.kernel,
        mesh=mesh,
        out_type=[
            jax.ShapeDtypeStruct((NC, N_PAD, D), jnp.float32),
            jax.ShapeDtypeStruct((NC, N_PAD), jnp.float32),
        ],
        scratch_types=[
            pltpu.VMEM((2, GRP, BLK), jnp.int32),
            pltpu.VMEM((2, GRP, BLK), jnp.int32),
            pltpu.VMEM((BLK, D), jnp.float32),
            pltpu.VMEM((BLK, D), jnp.float32),
            pltpu.VMEM((BLK,), jnp.float32),
            pltpu.VMEM((ROWS_PT,), jnp.float32),
            pltpu.VMEM_SHARED((N_PAD, D), jnp.float32),
            pltpu.VMEM_SHARED((N_PAD,), jnp.float32),
            pltpu.SemaphoreType.DMA,
            pltpu.SemaphoreType.DMA,
            pltpu.SemaphoreType.DMA,
            pltpu.SemaphoreType.DMA,
            pltpu.SemaphoreType.DMA,
            pltpu.SemaphoreType.DMA,
        ],
    )
    def body(h_hbm, src_hbm, dst_hbm, agg_out, deg_out,
             src_v, dst_v, rb0, rb1, ones_v, vec_v, agg_sh, deg_sh,
             sg0, sg1, ss0, ss1, sem_d, sem_i):
        cid = lax.axis_index("c")
        sid = lax.axis_index("s")
        wid = sid * NC + cid
        row0 = sid * ROWS_PT
        bufs = (rb0, rb1)
        gsems = (sg0, sg1)
        ssems = (ss0, ss1)

        def start_refill(g):
            p = g % 2
            pltpu.async_copy(src_hbm.at[wid, g], src_v.at[p], sem_i)
            pltpu.async_copy(dst_hbm.at[wid, g], dst_v.at[p], sem_i)

        def wait_refill(g):
            p = g % 2
            pltpu.make_async_copy(src_hbm.at[wid, 0], src_v.at[p],
                                  sem_i).wait()
            pltpu.make_async_copy(dst_hbm.at[wid, 0], dst_v.at[p],
                                  sem_i).wait()

        start_refill(0)

        zero16 = jnp.zeros((16,), jnp.float32)
        one16 = jnp.ones((16,), jnp.float32)

        def z_rows(i, _):
            r = i // (D // 16)
            c = (i % (D // 16)) * 16
            rb0[r, pl.ds(c, 16)] = zero16
            return 0
        lax.fori_loop(0, BLK * D // 16, z_rows, 0)

        def z_vec(i, _):
            vec_v[pl.ds(i * 16, 16)] = zero16
            return 0
        lax.fori_loop(0, ROWS_PT // 16, z_vec, 0)

        for i in range(BLK // 16):
            ones_v[pl.ds(i * 16, 16)] = one16

        for c in range(NCHUNK):
            pltpu.sync_copy(rb0, agg_sh.at[pl.ds(row0 + c * RCHUNK, RCHUNK)])
        pltpu.sync_copy(vec_v, deg_sh.at[pl.ds(row0, ROWS_PT)])
        plsc.subcore_barrier()

        def start_gather(p, i, k):
            pltpu.async_copy(h_hbm.at[src_v.at[p, i]], bufs[k], gsems[k])

        def wait_gather(k):
            pltpu.make_async_copy(h_hbm.at[src_v.at[0, 0]], bufs[k],
                                  gsems[k]).wait()

        def start_scatter(p, i, k):
            pltpu.async_copy(bufs[k], agg_sh.at[dst_v.at[p, i]], ssems[k],
                             add=True)

        def wait_scatter(k):
            pltpu.make_async_copy(bufs[k], agg_sh.at[dst_v.at[0, 0]],
                                  ssems[k]).wait()

        def start_deg(p, i):
            pltpu.async_copy(ones_v, deg_sh.at[dst_v.at[p, i]], sem_d,
                             add=True)

        def wait_deg():
            pltpu.make_async_copy(ones_v, deg_sh.at[dst_v.at[0, 0]],
                                  sem_d).wait()

        def group(g, _):
            p = g % 2
            wait_refill(g)
            pl.when(g + 1 < NGRP)(lambda: start_refill(g + 1))

            start_gather(p, 0, 0)
            start_gather(p, 1, 1)
            wait_gather(0)
            start_scatter(p, 0, 0)
            start_deg(p, 0)

            def dstep(j, _):
                for r in (0, 1):
                    i = 2 * j + 1 + r
                    k = (1, 0)[r]
                    wait_gather(k)
                    start_scatter(p, i, k)
                    wait_deg()
                    start_deg(p, i)
                    wait_scatter(1 - k)
                    start_gather(p, i + 1, 1 - k)
                return 0
            lax.fori_loop(0, (GRP - 3) // 2, dstep, 0)

            wait_gather(1)
            start_scatter(p, GRP - 2, 1)
            wait_deg()
            start_deg(p, GRP - 2)
            wait_scatter(0)
            start_gather(p, GRP - 1, 0)

            wait_gather(0)
            start_scatter(p, GRP - 1, 0)
            wait_deg()
            start_deg(p, GRP - 1)
            wait_scatter(1)

            wait_scatter(0)
            wait_deg()
            return 0
        lax.fori_loop(0, NGRP, group, 0)
        plsc.subcore_barrier()

        for c in range(NCHUNK):
            r = row0 + c * RCHUNK
            pltpu.sync_copy(agg_sh.at[pl.ds(r, RCHUNK)], rb0)
            pltpu.sync_copy(rb0, agg_out.at[cid, pl.ds(r, RCHUNK)])
        pltpu.sync_copy(deg_sh.at[pl.ds(row0, ROWS_PT)], vec_v)
        pltpu.sync_copy(vec_v, deg_out.at[cid, pl.ds(row0, ROWS_PT)])

    return body(h_pad, src3, dst3)


def _dense_body(relu, h_ref, agg_ref, deg_ref, ws_ref, wn_ref, b_ref, o_ref):
    hv = h_ref[...]
    a = agg_ref[0] + agg_ref[1]
    dg = deg_ref[0] + deg_ref[1]
    r = 1.0 / jnp.maximum(dg, 1.0)
    hn = a * r[:, None]
    o = (jnp.dot(hv, ws_ref[...], preferred_element_type=jnp.float32)
         + jnp.dot(hn, wn_ref[...], preferred_element_type=jnp.float32)
         + b_ref[...])
    if relu:
        o = jnp.maximum(o, 0.0)
    o_ref[...] = o


def _dense_layer(h_pad, agg_part, deg_part, w_self, w_neigh, b, relu):
    BN = 256
    grid = (N_PAD // BN,)
    return pl.pallas_call(
        functools.partial(_dense_body, relu),
        grid=grid,
        in_specs=[
            pl.BlockSpec((BN, D), lambda i: (i, 0)),
            pl.BlockSpec((NC, BN, D), lambda i: (0, i, 0)),
            pl.BlockSpec((NC, BN), lambda i: (0, i)),
            pl.BlockSpec((D, D), lambda i: (0, 0)),
            pl.BlockSpec((D, D), lambda i: (0, 0)),
            pl.BlockSpec((1, D), lambda i: (0, 0)),
        ],
        out_specs=pl.BlockSpec((BN, D), lambda i: (i, 0)),
        out_shape=jax.ShapeDtypeStruct((N_PAD, D), jnp.float32),
        compiler_params=pltpu.CompilerParams(
            dimension_semantics=("arbitrary",),
        ),
    )(h_pad, agg_part, deg_part, w_self, w_neigh, b.reshape(1, D))


def kernel(h, edge_index0, edge_index1, W_self0, W_neigh0, b0,
           W_self1, W_neigh1, b1):
    src0 = edge_index0[0].astype(jnp.int32).reshape(NW, NGRP, GRP, BLK)
    dst0 = edge_index0[1].astype(jnp.int32).reshape(NW, NGRP, GRP, BLK)
    src1 = edge_index1[0].astype(jnp.int32).reshape(NW, NGRP, GRP, BLK)
    dst1 = edge_index1[1].astype(jnp.int32).reshape(NW, NGRP, GRP, BLK)
    h_pad = jnp.pad(h, ((0, N_PAD - N), (0, 0)))

    agg0, deg0 = _sc_aggregate(h_pad, src0, dst0)
    x = _dense_layer(h_pad, agg0, deg0, W_self0, W_neigh0, b0, relu=True)
    agg1, deg1 = _sc_aggregate(x, src1, dst1)
    out = _dense_layer(x, agg1, deg1, W_self1, W_neigh1, b1, relu=False)
    return out[:N]


# speedup vs baseline: 1.1861x; 1.1861x over previous
import functools

import jax
import jax.numpy as jnp
from jax import lax
from jax.experimental import pallas as pl
from jax.experimental.pallas import tpu as pltpu
from jax.experimental.pallas import tpu_sc as plsc

N = 10000
E = 320000
D = 128
N_PAD = 10240

_info = plsc.get_sparse_core_info()
NC = _info.num_cores
NS = _info.num_subcores
NW = NC * NS
EPW = E // NW
BLK = 80
NBLK = EPW // BLK
GRP = 25
NGRP = NBLK // GRP
ROWS_PT = N_PAD // NS
RCHUNK = BLK
NCHUNK = ROWS_PT // RCHUNK


def _sc_aggregate(h_pad, src3, dst3):
    mesh = plsc.VectorSubcoreMesh(core_axis_name="c", subcore_axis_name="s")

    @functools.partial(
        pl.kernel,
        mesh=mesh,
        out_type=[
            jax.ShapeDtypeStruct((NC, N_PAD, D), jnp.float32),
            jax.ShapeDtypeStruct((NC, N_PAD), jnp.float32),
        ],
        scratch_types=[
            pltpu.VMEM((2, GRP, BLK), jnp.int32),
            pltpu.VMEM((2, GRP, BLK), jnp.int32),
            pltpu.VMEM((BLK, D), jnp.float32),
            pltpu.VMEM((BLK, D), jnp.float32),
            pltpu.VMEM((BLK,), jnp.float32),
            pltpu.VMEM((ROWS_PT,), jnp.float32),
            pltpu.VMEM_SHARED((N_PAD, D), jnp.float32),
            pltpu.VMEM_SHARED((N_PAD,), jnp.float32),
            pltpu.SemaphoreType.DMA,
            pltpu.SemaphoreType.DMA,
            pltpu.SemaphoreType.DMA,
            pltpu.SemaphoreType.DMA,
            pltpu.SemaphoreType.DMA,
            pltpu.SemaphoreType.DMA,
        ],
    )
    def body(h_hbm, src_hbm, dst_hbm, agg_out, deg_out,
             src_v, dst_v, rb0, rb1, ones_v, vec_v, agg_sh, deg_sh,
             sg0, sg1, ss0, ss1, sem_d, sem_i):
        cid = lax.axis_index("c")
        sid = lax.axis_index("s")
        wid = sid * NC + cid
        row0 = sid * ROWS_PT
        bufs = (rb0, rb1)
        gsems = (sg0, sg1)
        ssems = (ss0, ss1)

        def start_refill(g):
            p = g % 2
            pltpu.async_copy(src_hbm.at[wid, g], src_v.at[p], sem_i)
            pltpu.async_copy(dst_hbm.at[wid, g], dst_v.at[p], sem_i)

        def wait_refill(g):
            p = g % 2
            pltpu.make_async_copy(src_hbm.at[wid, 0], src_v.at[p],
                                  sem_i).wait()
            pltpu.make_async_copy(dst_hbm.at[wid, 0], dst_v.at[p],
                                  sem_i).wait()

        start_refill(0)

        zero16 = jnp.zeros((16,), jnp.float32)
        one16 = jnp.ones((16,), jnp.float32)

        def z_rows(i, _):
            r = i // (D // 16)
            c = (i % (D // 16)) * 16
            rb0[r, pl.ds(c, 16)] = zero16
            return 0
        lax.fori_loop(0, BLK * D // 16, z_rows, 0)

        def z_vec(i, _):
            vec_v[pl.ds(i * 16, 16)] = zero16
            return 0
        lax.fori_loop(0, ROWS_PT // 16, z_vec, 0)

        for i in range(BLK // 16):
            ones_v[pl.ds(i * 16, 16)] = one16

        for c in range(NCHUNK):
            pltpu.sync_copy(rb0, agg_sh.at[pl.ds(row0 + c * RCHUNK, RCHUNK)])
        pltpu.sync_copy(vec_v, deg_sh.at[pl.ds(row0, ROWS_PT)])
        plsc.subcore_barrier()

        def start_gather(p, i, k):
            pltpu.async_copy(h_hbm.at[src_v.at[p, i]], bufs[k], gsems[k])

        def wait_gather(k):
            pltpu.make_async_copy(h_hbm.at[src_v.at[0, 0]], bufs[k],
                                  gsems[k]).wait()

        def start_scatter(p, i, k):
            pltpu.async_copy(bufs[k], agg_sh.at[dst_v.at[p, i]], ssems[k],
                             add=True)

        def wait_scatter(k):
            pltpu.make_async_copy(bufs[k], agg_sh.at[dst_v.at[0, 0]],
                                  ssems[k]).wait()

        def start_deg(p, i):
            pltpu.async_copy(ones_v, deg_sh.at[dst_v.at[p, i]], sem_d,
                             add=True)

        def wait_deg():
            pltpu.make_async_copy(ones_v, deg_sh.at[dst_v.at[0, 0]],
                                  sem_d).wait()

        def group(g, _):
            p = g % 2
            wait_refill(g)
            pl.when(g + 1 < NGRP)(lambda: start_refill(g + 1))

            start_gather(p, 0, 0)
            start_gather(p, 1, 1)
            start_deg(p, 0)

            def dstep(j, _):
                for k in (0, 1):
                    i = 2 * j + k
                    wait_gather(k)
                    pltpu.sync_copy(bufs[k], agg_sh.at[dst_v.at[p, i]],
                                    add=True)
                    wait_deg()
                    start_deg(p, jnp.minimum(i + 1, GRP - 1))
                    start_gather(p, jnp.minimum(i + 2, GRP - 1), k)
                return 0
            lax.fori_loop(0, (GRP - 1) // 2, dstep, 0)

            wait_gather(0)
            pltpu.sync_copy(bufs[0], agg_sh.at[dst_v.at[p, GRP - 1]],
                            add=True)
            wait_deg()
            wait_gather(1)
            return 0
        lax.fori_loop(0, NGRP, group, 0)
        plsc.subcore_barrier()

        for c in range(NCHUNK):
            r = row0 + c * RCHUNK
            pltpu.sync_copy(agg_sh.at[pl.ds(r, RCHUNK)], rb0)
            pltpu.sync_copy(rb0, agg_out.at[cid, pl.ds(r, RCHUNK)])
        pltpu.sync_copy(deg_sh.at[pl.ds(row0, ROWS_PT)], vec_v)
        pltpu.sync_copy(vec_v, deg_out.at[cid, pl.ds(row0, ROWS_PT)])

    return body(h_pad, src3, dst3)


def _dense_body(relu, h_ref, agg_ref, deg_ref, ws_ref, wn_ref, b_ref, o_ref):
    hv = h_ref[...]
    a = agg_ref[0] + agg_ref[1]
    dg = deg_ref[0] + deg_ref[1]
    r = 1.0 / jnp.maximum(dg, 1.0)
    hn = a * r[:, None]
    o = (jnp.dot(hv, ws_ref[...], preferred_element_type=jnp.float32)
         + jnp.dot(hn, wn_ref[...], preferred_element_type=jnp.float32)
         + b_ref[...])
    if relu:
        o = jnp.maximum(o, 0.0)
    o_ref[...] = o


def _dense_layer(h_pad, agg_part, deg_part, w_self, w_neigh, b, relu):
    BN = 256
    grid = (N_PAD // BN,)
    return pl.pallas_call(
        functools.partial(_dense_body, relu),
        grid=grid,
        in_specs=[
            pl.BlockSpec((BN, D), lambda i: (i, 0)),
            pl.BlockSpec((NC, BN, D), lambda i: (0, i, 0)),
            pl.BlockSpec((NC, BN), lambda i: (0, i)),
            pl.BlockSpec((D, D), lambda i: (0, 0)),
            pl.BlockSpec((D, D), lambda i: (0, 0)),
            pl.BlockSpec((1, D), lambda i: (0, 0)),
        ],
        out_specs=pl.BlockSpec((BN, D), lambda i: (i, 0)),
        out_shape=jax.ShapeDtypeStruct((N_PAD, D), jnp.float32),
        compiler_params=pltpu.CompilerParams(
            dimension_semantics=("arbitrary",),
        ),
    )(h_pad, agg_part, deg_part, w_self, w_neigh, b.reshape(1, D))


def kernel(h, edge_index0, edge_index1, W_self0, W_neigh0, b0,
           W_self1, W_neigh1, b1):
    src0 = edge_index0[0].astype(jnp.int32).reshape(NW, NGRP, GRP, BLK)
    dst0 = edge_index0[1].astype(jnp.int32).reshape(NW, NGRP, GRP, BLK)
    src1 = edge_index1[0].astype(jnp.int32).reshape(NW, NGRP, GRP, BLK)
    dst1 = edge_index1[1].astype(jnp.int32).reshape(NW, NGRP, GRP, BLK)
    h_pad = jnp.pad(h, ((0, N_PAD - N), (0, 0)))

    agg0, deg0 = _sc_aggregate(h_pad, src0, dst0)
    x = _dense_layer(h_pad, agg0, deg0, W_self0, W_neigh0, b0, relu=True)
    agg1, deg1 = _sc_aggregate(x, src1, dst1)
    out = _dense_layer(x, agg1, deg1, W_self1, W_neigh1, b1, relu=False)
    return out[:N]


# speedup vs baseline: 1.3711x; 1.1560x over previous
import functools

import jax
import jax.numpy as jnp
from jax import lax
from jax.experimental import pallas as pl
from jax.experimental.pallas import tpu as pltpu
from jax.experimental.pallas import tpu_sc as plsc

N = 10000
E = 320000
D = 128
N_PAD = 10240

_info = plsc.get_sparse_core_info()
NC = _info.num_cores
NS = _info.num_subcores
NW = NC * NS
EPW = E // NW
BLK = 80
NBLK = EPW // BLK
GRP = 25
NGRP = NBLK // GRP
ROWS_PT = N_PAD // NS
RCHUNK = BLK
NCHUNK = ROWS_PT // RCHUNK


def _sc_aggregate(h_pad, src3, dst3):
    mesh = plsc.VectorSubcoreMesh(core_axis_name="c", subcore_axis_name="s")

    @functools.partial(
        pl.kernel,
        mesh=mesh,
        out_type=[
            jax.ShapeDtypeStruct((NC, N_PAD, D), jnp.float32),
            jax.ShapeDtypeStruct((NC, N_PAD), jnp.float32),
        ],
        scratch_types=[
            pltpu.VMEM((2, GRP, BLK), jnp.int32),
            pltpu.VMEM((2, GRP, BLK), jnp.int32),
            pltpu.VMEM((BLK, D), jnp.float32),
            pltpu.VMEM((BLK, D), jnp.float32),
            pltpu.VMEM((BLK,), jnp.float32),
            pltpu.VMEM((ROWS_PT,), jnp.float32),
            pltpu.VMEM_SHARED((N_PAD, D), jnp.float32),
            pltpu.VMEM_SHARED((N_PAD,), jnp.float32),
            pltpu.SemaphoreType.DMA,
            pltpu.SemaphoreType.DMA,
            pltpu.SemaphoreType.DMA,
            pltpu.SemaphoreType.DMA,
            pltpu.SemaphoreType.DMA,
            pltpu.SemaphoreType.DMA,
        ],
    )
    def body(h_hbm, src_hbm, dst_hbm, agg_out, deg_out,
             src_v, dst_v, rb0, rb1, ones_v, vec_v, agg_sh, deg_sh,
             sg0, sg1, ss0, ss1, sem_d, sem_i):
        cid = lax.axis_index("c")
        sid = lax.axis_index("s")
        wid = sid * NC + cid
        row0 = sid * ROWS_PT
        bufs = (rb0, rb1)
        gsems = (sg0, sg1)
        ssems = (ss0, ss1)

        def start_refill(g):
            p = g % 2
            pltpu.async_copy(src_hbm.at[wid, g], src_v.at[p], sem_i)
            pltpu.async_copy(dst_hbm.at[wid, g], dst_v.at[p], sem_i)

        def wait_refill(g):
            p = g % 2
            pltpu.make_async_copy(src_hbm.at[wid, 0], src_v.at[p],
                                  sem_i).wait()
            pltpu.make_async_copy(dst_hbm.at[wid, 0], dst_v.at[p],
                                  sem_i).wait()

        start_refill(0)

        zero16 = jnp.zeros((16,), jnp.float32)
        one16 = jnp.ones((16,), jnp.float32)

        def z_rows(i, _):
            r = i // (D // 16)
            c = (i % (D // 16)) * 16
            rb0[r, pl.ds(c, 16)] = zero16
            return 0
        lax.fori_loop(0, BLK * D // 16, z_rows, 0)

        def z_vec(i, _):
            vec_v[pl.ds(i * 16, 16)] = zero16
            return 0
        lax.fori_loop(0, ROWS_PT // 16, z_vec, 0)

        for i in range(BLK // 16):
            ones_v[pl.ds(i * 16, 16)] = one16

        for c in range(NCHUNK):
            pltpu.sync_copy(rb0, agg_sh.at[pl.ds(row0 + c * RCHUNK, RCHUNK)])
        pltpu.sync_copy(vec_v, deg_sh.at[pl.ds(row0, ROWS_PT)])
        plsc.subcore_barrier()

        def start_gather(p, i, k):
            pltpu.async_copy(h_hbm.at[src_v.at[p, i]], bufs[k], gsems[k])

        def wait_gather(k):
            pltpu.make_async_copy(h_hbm.at[src_v.at[0, 0]], bufs[k],
                                  gsems[k]).wait()

        def start_scatter(p, i, k):
            pltpu.async_copy(bufs[k], agg_sh.at[dst_v.at[p, i]], ssems[k],
                             add=True)

        def wait_scatter(k):
            pltpu.make_async_copy(bufs[k], agg_sh.at[dst_v.at[0, 0]],
                                  ssems[k]).wait()

        def start_deg(p, i):
            pltpu.async_copy(ones_v, deg_sh.at[dst_v.at[p, i]], sem_d,
                             add=True)

        def wait_deg():
            pltpu.make_async_copy(ones_v, deg_sh.at[dst_v.at[0, 0]],
                                  sem_d).wait()

        def group(g, _):
            p = g % 2
            wait_refill(g)
            pl.when(g + 1 < NGRP)(lambda: start_refill(g + 1))

            start_gather(p, 0, 0)
            start_gather(p, 1, 1)
            start_deg(p, 0)

            def dstep(j, _):
                for k in (0, 1):
                    i = 2 * j + k
                    wait_gather(k)
                    pltpu.sync_copy(bufs[k], agg_sh.at[dst_v.at[p, i]],
                                    add=True)
                    wait_deg()
                    start_deg(p, jnp.minimum(i + 1, GRP - 1))
                    start_gather(p, jnp.minimum(i + 2, GRP - 1), k)
                return 0
            lax.fori_loop(0, (GRP - 1) // 2, dstep, 0)

            wait_gather(0)
            pltpu.sync_copy(bufs[0], agg_sh.at[dst_v.at[p, GRP - 1]],
                            add=True)
            wait_deg()
            wait_gather(1)
            return 0
        lax.fori_loop(0, NGRP, group, 0)
        plsc.subcore_barrier()

        for c in range(NCHUNK):
            r = row0 + c * RCHUNK
            pltpu.sync_copy(agg_sh.at[pl.ds(r, RCHUNK)], rb0)
            pltpu.sync_copy(rb0, agg_out.at[cid, pl.ds(r, RCHUNK)])
        pltpu.sync_copy(deg_sh.at[pl.ds(row0, ROWS_PT)], vec_v)
        pltpu.sync_copy(vec_v, deg_out.at[cid, pl.ds(row0, ROWS_PT)])

    return body(h_pad, src3, dst3)


def _dense_body(relu, h_ref, agg_ref, deg_ref, ws_ref, wn_ref, b_ref, o_ref):
    hv = h_ref[...]
    a = agg_ref[0, :N] + agg_ref[1, :N]
    dg = deg_ref[0, :N] + deg_ref[1, :N]
    r = 1.0 / jnp.maximum(dg, 1.0)
    hn = a * r[:, None]
    o = (jnp.dot(hv, ws_ref[...], preferred_element_type=jnp.float32)
         + jnp.dot(hn, wn_ref[...], preferred_element_type=jnp.float32)
         + b_ref[...])
    if relu:
        o = jnp.maximum(o, 0.0)
    o_ref[...] = o


def _dense_layer(h, agg_part, deg_part, w_self, w_neigh, b, relu):
    return pl.pallas_call(
        functools.partial(_dense_body, relu),
        out_shape=jax.ShapeDtypeStruct((N, D), jnp.float32),
    )(h, agg_part, deg_part, w_self, w_neigh, b.reshape(1, D))


def kernel(h, edge_index0, edge_index1, W_self0, W_neigh0, b0,
           W_self1, W_neigh1, b1):
    src0 = edge_index0[0].astype(jnp.int32).reshape(NW, NGRP, GRP, BLK)
    dst0 = edge_index0[1].astype(jnp.int32).reshape(NW, NGRP, GRP, BLK)
    src1 = edge_index1[0].astype(jnp.int32).reshape(NW, NGRP, GRP, BLK)
    dst1 = edge_index1[1].astype(jnp.int32).reshape(NW, NGRP, GRP, BLK)

    agg0, deg0 = _sc_aggregate(h, src0, dst0)
    x = _dense_layer(h, agg0, deg0, W_self0, W_neigh0, b0, relu=True)
    agg1, deg1 = _sc_aggregate(x, src1, dst1)
    out = _dense_layer(x, agg1, deg1, W_self1, W_neigh1, b1, relu=False)
    return out


# speedup vs baseline: 1.4146x; 1.0317x over previous
import functools

import jax
import jax.numpy as jnp
from jax import lax
from jax.experimental import pallas as pl
from jax.experimental.pallas import tpu as pltpu
from jax.experimental.pallas import tpu_sc as plsc

N = 10000
E = 320000
D = 128
N_PAD = 10240

_info = plsc.get_sparse_core_info()
NC = _info.num_cores
NS = _info.num_subcores
NW = NC * NS
EPW = E // NW
BLK = 80
NBLK = EPW // BLK
GRP = 25
NGRP = NBLK // GRP
ROWS_PT = N_PAD // NS
RCHUNK = BLK
NCHUNK = ROWS_PT // RCHUNK


def _sc_aggregate(h_pad, src3, dst3):
    mesh = plsc.VectorSubcoreMesh(core_axis_name="c", subcore_axis_name="s")

    @functools.partial(
        pl.kernel,
        mesh=mesh,
        out_type=[
            jax.ShapeDtypeStruct((NC, N_PAD, D), jnp.float32),
            jax.ShapeDtypeStruct((NC, N_PAD), jnp.float32),
        ],
        scratch_types=[
            pltpu.VMEM((2, GRP, BLK), jnp.int32),
            pltpu.VMEM((2, GRP, BLK), jnp.int32),
            pltpu.VMEM((BLK, D), jnp.float32),
            pltpu.VMEM((BLK, D), jnp.float32),
            pltpu.VMEM((BLK,), jnp.float32),
            pltpu.VMEM((ROWS_PT,), jnp.float32),
            pltpu.VMEM_SHARED((N_PAD, D), jnp.float32),
            pltpu.VMEM_SHARED((N_PAD,), jnp.float32),
            pltpu.SemaphoreType.DMA,
            pltpu.SemaphoreType.DMA,
            pltpu.SemaphoreType.DMA,
            pltpu.SemaphoreType.DMA,
            pltpu.SemaphoreType.DMA,
            pltpu.SemaphoreType.DMA,
        ],
    )
    def body(h_hbm, src_hbm, dst_hbm, agg_out, deg_out,
             src_v, dst_v, rb0, rb1, ones_v, vec_v, agg_sh, deg_sh,
             sg0, sg1, ss0, ss1, sem_d, sem_i):
        cid = lax.axis_index("c")
        sid = lax.axis_index("s")
        wid = sid * NC + cid
        row0 = sid * ROWS_PT
        bufs = (rb0, rb1)
        gsems = (sg0, sg1)
        ssems = (ss0, ss1)

        def start_refill(g):
            p = g % 2
            pltpu.async_copy(src_hbm.at[wid, g], src_v.at[p], sem_i)
            pltpu.async_copy(dst_hbm.at[wid, g], dst_v.at[p], sem_i)

        def wait_refill(g):
            p = g % 2
            pltpu.make_async_copy(src_hbm.at[wid, 0], src_v.at[p],
                                  sem_i).wait()
            pltpu.make_async_copy(dst_hbm.at[wid, 0], dst_v.at[p],
                                  sem_i).wait()

        start_refill(0)

        zero16 = jnp.zeros((16,), jnp.float32)
        one16 = jnp.ones((16,), jnp.float32)

        def z_rows(i, _):
            for c in range(D // 16):
                rb0[i, pl.ds(c * 16, 16)] = zero16
            return 0
        lax.fori_loop(0, BLK, z_rows, 0)

        def z_vec(i, _):
            for c in range(8):
                vec_v[pl.ds(i * 128 + c * 16, 16)] = zero16
            return 0
        lax.fori_loop(0, ROWS_PT // 128, z_vec, 0)

        for i in range(BLK // 16):
            ones_v[pl.ds(i * 16, 16)] = one16

        for c in range(NCHUNK):
            pltpu.async_copy(rb0, agg_sh.at[pl.ds(row0 + c * RCHUNK, RCHUNK)],
                             ss0)
        pltpu.async_copy(vec_v, deg_sh.at[pl.ds(row0, ROWS_PT)], ss1)
        for c in range(NCHUNK):
            pltpu.make_async_copy(rb0, agg_sh.at[pl.ds(row0, RCHUNK)],
                                  ss0).wait()
        pltpu.make_async_copy(vec_v, deg_sh.at[pl.ds(row0, ROWS_PT)],
                              ss1).wait()
        plsc.subcore_barrier()

        def start_gather(p, i, k):
            pltpu.async_copy(h_hbm.at[src_v.at[p, i]], bufs[k], gsems[k])

        def wait_gather(k):
            pltpu.make_async_copy(h_hbm.at[src_v.at[0, 0]], bufs[k],
                                  gsems[k]).wait()

        def start_scatter(p, i, k):
            pltpu.async_copy(bufs[k], agg_sh.at[dst_v.at[p, i]], ssems[k],
                             add=True)

        def wait_scatter(k):
            pltpu.make_async_copy(bufs[k], agg_sh.at[dst_v.at[0, 0]],
                                  ssems[k]).wait()

        def start_deg(p, i):
            pltpu.async_copy(ones_v, deg_sh.at[dst_v.at[p, i]], sem_d,
                             add=True)

        def wait_deg():
            pltpu.make_async_copy(ones_v, deg_sh.at[dst_v.at[0, 0]],
                                  sem_d).wait()

        def group(g, _):
            p = g % 2
            wait_refill(g)
            pl.when(g + 1 < NGRP)(lambda: start_refill(g + 1))

            start_gather(p, 0, 0)
            start_gather(p, 1, 1)
            start_deg(p, 0)

            def dstep(j, _):
                for k in (0, 1):
                    i = 2 * j + k
                    wait_gather(k)
                    pltpu.sync_copy(bufs[k], agg_sh.at[dst_v.at[p, i]],
                                    add=True)
                    wait_deg()
                    start_deg(p, jnp.minimum(i + 1, GRP - 1))
                    start_gather(p, jnp.minimum(i + 2, GRP - 1), k)
                return 0
            lax.fori_loop(0, (GRP - 1) // 2, dstep, 0)

            wait_gather(0)
            pltpu.sync_copy(bufs[0], agg_sh.at[dst_v.at[p, GRP - 1]],
                            add=True)
            wait_deg()
            wait_gather(1)
            return 0
        lax.fori_loop(0, NGRP, group, 0)
        plsc.subcore_barrier()

        pltpu.async_copy(deg_sh.at[pl.ds(row0, ROWS_PT)], vec_v, sem_d)
        rbs = (rb0, rb1)

        def win(c, k):
            pltpu.async_copy(agg_sh.at[pl.ds(row0 + c * RCHUNK, RCHUNK)],
                             rbs[k], gsems[k])

        def wout(c, k):
            pltpu.async_copy(rbs[k],
                             agg_out.at[cid, pl.ds(row0 + c * RCHUNK, RCHUNK)],
                             ssems[k])

        win(0, 0)
        win(1, 1)
        for c in range(NCHUNK):
            k = c % 2
            pltpu.make_async_copy(agg_sh.at[pl.ds(row0, RCHUNK)], rbs[k],
                                  gsems[k]).wait()
            wout(c, k)
            if c + 2 < NCHUNK:
                pltpu.make_async_copy(
                    rbs[k], agg_out.at[cid, pl.ds(row0, RCHUNK)],
                    ssems[k]).wait()
                win(c + 2, k)
        pltpu.make_async_copy(deg_sh.at[pl.ds(row0, ROWS_PT)], vec_v,
                              sem_d).wait()
        pltpu.async_copy(vec_v, deg_out.at[cid, pl.ds(row0, ROWS_PT)], sem_d)
        for k in (0, 1):
            pltpu.make_async_copy(rbs[k], agg_out.at[cid, pl.ds(row0, RCHUNK)],
                                  ssems[k]).wait()
        pltpu.make_async_copy(vec_v, deg_out.at[cid, pl.ds(row0, ROWS_PT)],
                              sem_d).wait()

    return body(h_pad, src3, dst3)


def _dense_body(relu, h_ref, agg_ref, deg_ref, ws_ref, wn_ref, b_ref, o_ref):
    hv = h_ref[...]
    a = agg_ref[0, :N] + agg_ref[1, :N]
    dg = deg_ref[0, :N] + deg_ref[1, :N]
    r = 1.0 / jnp.maximum(dg, 1.0)
    hn = a * r[:, None]
    o = (jnp.dot(hv, ws_ref[...], preferred_element_type=jnp.float32)
         + jnp.dot(hn, wn_ref[...], preferred_element_type=jnp.float32)
         + b_ref[...])
    if relu:
        o = jnp.maximum(o, 0.0)
    o_ref[...] = o


def _dense_layer(h, agg_part, deg_part, w_self, w_neigh, b, relu):
    return pl.pallas_call(
        functools.partial(_dense_body, relu),
        out_shape=jax.ShapeDtypeStruct((N, D), jnp.float32),
    )(h, agg_part, deg_part, w_self, w_neigh, b.reshape(1, D))


def kernel(h, edge_index0, edge_index1, W_self0, W_neigh0, b0,
           W_self1, W_neigh1, b1):
    src0 = edge_index0[0].astype(jnp.int32).reshape(NW, NGRP, GRP, BLK)
    dst0 = edge_index0[1].astype(jnp.int32).reshape(NW, NGRP, GRP, BLK)
    src1 = edge_index1[0].astype(jnp.int32).reshape(NW, NGRP, GRP, BLK)
    dst1 = edge_index1[1].astype(jnp.int32).reshape(NW, NGRP, GRP, BLK)

    agg0, deg0 = _sc_aggregate(h, src0, dst0)
    x = _dense_layer(h, agg0, deg0, W_self0, W_neigh0, b0, relu=True)
    agg1, deg1 = _sc_aggregate(x, src1, dst1)
    out = _dense_layer(x, agg1, deg1, W_self1, W_neigh1, b1, relu=False)
    return out


# speedup vs baseline: 1.4704x; 1.0395x over previous
import functools

import jax
import jax.numpy as jnp
from jax import lax
from jax.experimental import pallas as pl
from jax.experimental.pallas import tpu as pltpu
from jax.experimental.pallas import tpu_sc as plsc

N = 10000
E = 320000
D = 128
N_PAD = 10240

_info = plsc.get_sparse_core_info()
NC = _info.num_cores
NS = _info.num_subcores
NW = NC * NS
EPW = E // NW
BLK = 80
NBLK = EPW // BLK
GRP = 25
NGRP = NBLK // GRP
ROWS_PT = N_PAD // NS
RCHUNK = BLK
NCHUNK = ROWS_PT // RCHUNK


def _sc_aggregate(h_pad, edge5):
    mesh = plsc.VectorSubcoreMesh(core_axis_name="c", subcore_axis_name="s")

    @functools.partial(
        pl.kernel,
        mesh=mesh,
        out_type=[
            jax.ShapeDtypeStruct((NC, N_PAD, D), jnp.float32),
            jax.ShapeDtypeStruct((NC, N_PAD), jnp.float32),
        ],
        scratch_types=[
            pltpu.VMEM((2, GRP, BLK), jnp.int32),
            pltpu.VMEM((2, GRP, BLK), jnp.int32),
            pltpu.VMEM((BLK, D), jnp.float32),
            pltpu.VMEM((BLK, D), jnp.float32),
            pltpu.VMEM((BLK,), jnp.float32),
            pltpu.VMEM((ROWS_PT,), jnp.float32),
            pltpu.VMEM_SHARED((N_PAD, D), jnp.float32),
            pltpu.VMEM_SHARED((N_PAD,), jnp.float32),
            pltpu.SemaphoreType.DMA,
            pltpu.SemaphoreType.DMA,
            pltpu.SemaphoreType.DMA,
            pltpu.SemaphoreType.DMA,
            pltpu.SemaphoreType.DMA,
            pltpu.SemaphoreType.DMA,
        ],
    )
    def body(h_hbm, edge_hbm, agg_out, deg_out,
             src_v, dst_v, rb0, rb1, ones_v, vec_v, agg_sh, deg_sh,
             sg0, sg1, ss0, ss1, sem_d, sem_i):
        cid = lax.axis_index("c")
        sid = lax.axis_index("s")
        wid = sid * NC + cid
        row0 = sid * ROWS_PT
        bufs = (rb0, rb1)
        gsems = (sg0, sg1)
        ssems = (ss0, ss1)

        def start_refill(g):
            p = g % 2
            pltpu.async_copy(edge_hbm.at[0, wid, g], src_v.at[p], sem_i)
            pltpu.async_copy(edge_hbm.at[1, wid, g], dst_v.at[p], sem_i)

        def wait_refill(g):
            p = g % 2
            pltpu.make_async_copy(edge_hbm.at[0, wid, 0], src_v.at[p],
                                  sem_i).wait()
            pltpu.make_async_copy(edge_hbm.at[1, wid, 0], dst_v.at[p],
                                  sem_i).wait()

        start_refill(0)

        zero16 = jnp.zeros((16,), jnp.float32)
        one16 = jnp.ones((16,), jnp.float32)

        def z_rows(i, _):
            for c in range(D // 16):
                rb0[i, pl.ds(c * 16, 16)] = zero16
            return 0
        lax.fori_loop(0, BLK, z_rows, 0)

        def z_vec(i, _):
            for c in range(8):
                vec_v[pl.ds(i * 128 + c * 16, 16)] = zero16
            return 0
        lax.fori_loop(0, ROWS_PT // 128, z_vec, 0)

        for i in range(BLK // 16):
            ones_v[pl.ds(i * 16, 16)] = one16

        for c in range(NCHUNK):
            pltpu.async_copy(rb0, agg_sh.at[pl.ds(row0 + c * RCHUNK, RCHUNK)],
                             ss0)
        pltpu.async_copy(vec_v, deg_sh.at[pl.ds(row0, ROWS_PT)], ss1)
        for c in range(NCHUNK):
            pltpu.make_async_copy(rb0, agg_sh.at[pl.ds(row0, RCHUNK)],
                                  ss0).wait()
        pltpu.make_async_copy(vec_v, deg_sh.at[pl.ds(row0, ROWS_PT)],
                              ss1).wait()
        plsc.subcore_barrier()

        def start_gather(p, i, k):
            pltpu.async_copy(h_hbm.at[src_v.at[p, i]], bufs[k], gsems[k])

        def wait_gather(k):
            pltpu.make_async_copy(h_hbm.at[src_v.at[0, 0]], bufs[k],
                                  gsems[k]).wait()

        def start_scatter(p, i, k):
            pltpu.async_copy(bufs[k], agg_sh.at[dst_v.at[p, i]], ssems[k],
                             add=True)

        def wait_scatter(k):
            pltpu.make_async_copy(bufs[k], agg_sh.at[dst_v.at[0, 0]],
                                  ssems[k]).wait()

        def start_deg(p, i):
            pltpu.async_copy(ones_v, deg_sh.at[dst_v.at[p, i]], sem_d,
                             add=True)

        def wait_deg():
            pltpu.make_async_copy(ones_v, deg_sh.at[dst_v.at[0, 0]],
                                  sem_d).wait()

        def group(g, _):
            p = g % 2
            wait_refill(g)
            pl.when(g + 1 < NGRP)(lambda: start_refill(g + 1))

            start_gather(p, 0, 0)
            start_gather(p, 1, 1)
            start_deg(p, 0)

            def dstep(j, _):
                for k in (0, 1):
                    i = 2 * j + k
                    wait_gather(k)
                    pltpu.sync_copy(bufs[k], agg_sh.at[dst_v.at[p, i]],
                                    add=True)
                    wait_deg()
                    start_deg(p, jnp.minimum(i + 1, GRP - 1))
                    start_gather(p, jnp.minimum(i + 2, GRP - 1), k)
                return 0
            lax.fori_loop(0, (GRP - 1) // 2, dstep, 0)

            wait_gather(0)
            pltpu.sync_copy(bufs[0], agg_sh.at[dst_v.at[p, GRP - 1]],
                            add=True)
            wait_deg()
            wait_gather(1)
            return 0
        lax.fori_loop(0, NGRP, group, 0)
        plsc.subcore_barrier()

        pltpu.async_copy(deg_sh.at[pl.ds(row0, ROWS_PT)], vec_v, sem_d)
        rbs = (rb0, rb1)

        def win(c, k):
            pltpu.async_copy(agg_sh.at[pl.ds(row0 + c * RCHUNK, RCHUNK)],
                             rbs[k], gsems[k])

        def wout(c, k):
            pltpu.async_copy(rbs[k],
                             agg_out.at[cid, pl.ds(row0 + c * RCHUNK, RCHUNK)],
                             ssems[k])

        win(0, 0)
        win(1, 1)
        for c in range(NCHUNK):
            k = c % 2
            pltpu.make_async_copy(agg_sh.at[pl.ds(row0, RCHUNK)], rbs[k],
                                  gsems[k]).wait()
            wout(c, k)
            if c + 2 < NCHUNK:
                pltpu.make_async_copy(
                    rbs[k], agg_out.at[cid, pl.ds(row0, RCHUNK)],
                    ssems[k]).wait()
                win(c + 2, k)
        pltpu.make_async_copy(deg_sh.at[pl.ds(row0, ROWS_PT)], vec_v,
                              sem_d).wait()
        pltpu.async_copy(vec_v, deg_out.at[cid, pl.ds(row0, ROWS_PT)], sem_d)
        for k in (0, 1):
            pltpu.make_async_copy(rbs[k], agg_out.at[cid, pl.ds(row0, RCHUNK)],
                                  ssems[k]).wait()
        pltpu.make_async_copy(vec_v, deg_out.at[cid, pl.ds(row0, ROWS_PT)],
                              sem_d).wait()

    return body(h_pad, edge5)


def _dense_body(relu, h_ref, agg_ref, deg_ref, ws_ref, wn_ref, b_ref, o_ref):
    hv = h_ref[...]
    a = agg_ref[0, :N] + agg_ref[1, :N]
    dg = deg_ref[0, :N] + deg_ref[1, :N]
    r = 1.0 / jnp.maximum(dg, 1.0)
    hn = a * r[:, None]
    o = (jnp.dot(hv, ws_ref[...], preferred_element_type=jnp.float32)
         + jnp.dot(hn, wn_ref[...], preferred_element_type=jnp.float32)
         + b_ref[...])
    if relu:
        o = jnp.maximum(o, 0.0)
    o_ref[...] = o


def _dense_layer(h, agg_part, deg_part, w_self, w_neigh, b, relu):
    return pl.pallas_call(
        functools.partial(_dense_body, relu),
        out_shape=jax.ShapeDtypeStruct((N, D), jnp.float32),
    )(h, agg_part, deg_part, w_self, w_neigh, b.reshape(1, D))


def kernel(h, edge_index0, edge_index1, W_self0, W_neigh0, b0,
           W_self1, W_neigh1, b1):
    e0 = edge_index0.astype(jnp.int32).reshape(2, NW, NGRP, GRP, BLK)
    e1 = edge_index1.astype(jnp.int32).reshape(2, NW, NGRP, GRP, BLK)

    agg0, deg0 = _sc_aggregate(h, e0)
    x = _dense_layer(h, agg0, deg0, W_self0, W_neigh0, b0, relu=True)
    agg1, deg1 = _sc_aggregate(x, e1)
    out = _dense_layer(x, agg1, deg1, W_self1, W_neigh1, b1, relu=False)
    return out
